# baseline (speedup 1.0000x reference)
import functools

import jax
import jax.numpy as jnp
from jax import lax
from jax.experimental import pallas as pl
from jax.experimental.pallas import tpu as pltpu
from jax.experimental.pallas import tpu_sc as plsc

_CH = 128


@functools.lru_cache(maxsize=None)
def _make_gather(v, d, nw, per_w):
    n_ch = per_w // _CH
    n_rows = nw * per_w
    mesh = plsc.VectorSubcoreMesh(core_axis_name="c", subcore_axis_name="s")

    @functools.partial(
        pl.kernel,
        out_type=jax.ShapeDtypeStruct((n_rows, d), jnp.float32),
        mesh=mesh,
        compiler_params=pltpu.CompilerParams(use_tc_tiling_on_sc=False),
        scratch_types=[
            pltpu.VMEM((n_ch, _CH), jnp.int32),
            pltpu.VMEM((_CH, d), jnp.float32),
            pltpu.VMEM((_CH, d), jnp.float32),
            pltpu.SemaphoreType.DMA,
            pltpu.SemaphoreType.DMA,
        ],
    )
    def gather_kernel(idx_hbm, table_hbm, out_hbm, idx_v, buf0, buf1, sem0, sem1):
        wid = lax.axis_index("s") * 2 + lax.axis_index("c")
        base = wid * per_w
        pltpu.sync_copy(idx_hbm.at[wid], idx_v)
        pltpu.async_copy(table_hbm.at[idx_v.at[0]], buf0, sem0)

        @pl.loop(0, n_ch, step=2)
        def _(g):
            pltpu.async_copy(table_hbm.at[idx_v.at[g + 1]], buf1, sem1)
            pltpu.make_async_copy(table_hbm.at[idx_v.at[g]], buf0, sem0).wait()
            pltpu.sync_copy(buf0, out_hbm.at[pl.ds(base + g * _CH, _CH)])

            @pl.when(g + 2 < n_ch)
            def _start_next():
                pltpu.async_copy(table_hbm.at[idx_v.at[g + 2]], buf0, sem0)

            pltpu.make_async_copy(table_hbm.at[idx_v.at[g + 1]], buf1, sem1).wait()
            pltpu.sync_copy(buf1, out_hbm.at[pl.ds(base + (g + 1) * _CH, _CH)])

    return gather_kernel


def kernel(indexseq, table):
    b, s = indexseq.shape
    v, d = table.shape
    nw = 32
    n_rows = s * b
    per_w = n_rows // nw
    idx3 = jnp.transpose(indexseq.astype(jnp.int32)).reshape(nw, per_w // _CH, _CH)
    out_flat = _make_gather(v, d, nw, per_w)(idx3, table)
    return out_flat.reshape(s, b, d)


# speedup vs baseline: 1.0226x; 1.0226x over previous
import functools

import jax
import jax.numpy as jnp
from jax import lax
from jax.experimental import pallas as pl
from jax.experimental.pallas import tpu as pltpu
from jax.experimental.pallas import tpu_sc as plsc

_CH = 128
_NBUF = 4


@functools.lru_cache(maxsize=None)
def _make_gather(v, d, nw, per_w):
    n_ch = per_w // _CH
    n_rows = nw * per_w
    mesh = plsc.VectorSubcoreMesh(core_axis_name="c", subcore_axis_name="s")

    @functools.partial(
        pl.kernel,
        out_type=jax.ShapeDtypeStruct((n_rows, d), jnp.float32),
        mesh=mesh,
        compiler_params=pltpu.CompilerParams(use_tc_tiling_on_sc=False),
        scratch_types=[
            pltpu.VMEM((n_ch, _CH), jnp.int32),
            [pltpu.VMEM((_CH, d), jnp.float32) for _ in range(_NBUF)],
            [pltpu.SemaphoreType.DMA for _ in range(_NBUF)],
        ],
    )
    def gather_kernel(idx_hbm, table_hbm, out_hbm, idx_v, bufs, sems):
        wid = lax.axis_index("s") * 2 + lax.axis_index("c")
        base = wid * per_w
        pltpu.sync_copy(idx_hbm.at[wid], idx_v)
        for p in range(_NBUF):
            pltpu.async_copy(table_hbm.at[idx_v.at[p]], bufs[p], sems[p])

        @pl.loop(0, n_ch, step=_NBUF)
        def _(i):
            for p in range(_NBUF):
                g = i + p
                pltpu.make_async_copy(
                    table_hbm.at[idx_v.at[g]], bufs[p], sems[p]).wait()
                pltpu.sync_copy(bufs[p], out_hbm.at[pl.ds(base + g * _CH, _CH)])

                @pl.when(g + _NBUF < n_ch)
                def _refill():
                    pltpu.async_copy(
                        table_hbm.at[idx_v.at[g + _NBUF]], bufs[p], sems[p])

    return gather_kernel


def kernel(indexseq, table):
    b, s = indexseq.shape
    v, d = table.shape
    nw = 32
    n_rows = s * b
    per_w = n_rows // nw
    idx3 = jnp.transpose(indexseq.astype(jnp.int32)).reshape(nw, per_w // _CH, _CH)
    out_flat = _make_gather(v, d, nw, per_w)(idx3, table)
    return out_flat.reshape(s, b, d)
